# baseline (speedup 1.0000x reference)
import functools

import jax
import jax.numpy as jnp
from jax import lax
from jax.experimental import pallas as pl
from jax.experimental.pallas import tpu as pltpu
from jax.experimental.pallas import tpu_sc as plsc

N_FIELDS = 26
BATCH = 16384
VOCAB = 100000
EMBED_DIM = 32

_INFO = plsc.get_sparse_core_info()
NC, NS, L = _INFO.num_cores, _INFO.num_subcores, _INFO.num_lanes
NW = NC * NS
F_HALF = N_FIELDS // 2


def _sc_body(tables_tr, xidx3, out_t, vec, orow, ichunk):
    wid = lax.axis_index("s") * NC + lax.axis_index("c")

    def field_step(f, carry):
        pltpu.sync_copy(tables_tr.at[f, wid], vec)
        for h in range(2):
            pltpu.sync_copy(xidx3.at[f, pl.ds(h * 64, 64)], ichunk)

            def body(k, c):
                for j in range(8):
                    iv = ichunk[k, pl.ds(j * L, L)]
                    vals = plsc.load_gather(vec, [iv])
                    orow[pl.ds(h * 8192 + k * 128 + j * L, L)] = vals
                return c

            lax.fori_loop(0, 64, body, 0)
        pltpu.sync_copy(orow, out_t.at[f * EMBED_DIM + wid])
        return carry

    lax.fori_loop(0, F_HALF, field_step, 0)


def _half(tables_tr_half, xidx3_half):
    mesh = plsc.VectorSubcoreMesh(core_axis_name="c", subcore_axis_name="s")
    fn = pl.kernel(
        _sc_body,
        out_type=jax.ShapeDtypeStruct((F_HALF * EMBED_DIM, BATCH),
                                      jnp.float32),
        mesh=mesh,
        scratch_types=[
            pltpu.VMEM((VOCAB,), jnp.float32),
            pltpu.VMEM((BATCH,), jnp.float32),
            pltpu.VMEM((64, 128), jnp.int32),
        ],
        compiler_params=pltpu.CompilerParams(use_tc_tiling_on_sc=False,
                                             needs_layout_passes=False),
    )
    return fn(tables_tr_half, xidx3_half)


def kernel(x_cat, tables):
    tables_tr = tables.transpose(0, 2, 1)
    xidx3 = x_cat.astype(jnp.int32).reshape(N_FIELDS, BATCH // 128, 128)
    out_a = _half(tables_tr[:F_HALF], xidx3[:F_HALF])
    out_b = _half(tables_tr[F_HALF:], xidx3[F_HALF:])
    return jnp.concatenate([out_a, out_b], axis=0).T


# speedup vs baseline: 1.1419x; 1.1419x over previous
import jax
import jax.numpy as jnp
from jax import lax
from jax.experimental import pallas as pl
from jax.experimental.pallas import tpu as pltpu
from jax.experimental.pallas import tpu_sc as plsc

N_FIELDS = 26
BATCH = 16384
VOCAB = 100000
EMBED_DIM = 32

_INFO = plsc.get_sparse_core_info()
NC, NS, L = _INFO.num_cores, _INFO.num_subcores, _INFO.num_lanes
NW = NC * NS


def _sc_body(tables_tr, xidx3, out_t, vec, orow, ichunk):
    wid = lax.axis_index("s") * NC + lax.axis_index("c")

    def field_step(f, carry):
        pltpu.sync_copy(tables_tr.at[f, wid], vec)
        for h in range(2):
            pltpu.sync_copy(xidx3.at[f, pl.ds(h * 64, 64)], ichunk)

            def body(k, c):
                for j in range(8):
                    iv = ichunk[k, pl.ds(j * L, L)]
                    vals = plsc.load_gather(vec, [iv])
                    orow[pl.ds(h * 8192 + k * 128 + j * L, L)] = vals
                return c

            lax.fori_loop(0, 64, body, 0)
        pltpu.sync_copy(orow, out_t.at[f * EMBED_DIM + wid])
        return carry

    lax.fori_loop(0, N_FIELDS, field_step, 0)


def kernel(x_cat, tables):
    tables_tr = tables.transpose(0, 2, 1)
    xidx3 = x_cat.astype(jnp.int32).reshape(N_FIELDS, BATCH // 128, 128)
    mesh = plsc.VectorSubcoreMesh(core_axis_name="c", subcore_axis_name="s")
    fn = pl.kernel(
        _sc_body,
        out_type=jax.ShapeDtypeStruct((N_FIELDS * EMBED_DIM, BATCH),
                                      jnp.float32),
        mesh=mesh,
        scratch_types=[
            pltpu.VMEM((VOCAB,), jnp.float32),
            pltpu.VMEM((BATCH,), jnp.float32),
            pltpu.VMEM((64, 128), jnp.int32),
        ],
        compiler_params=pltpu.CompilerParams(use_tc_tiling_on_sc=False,
                                             needs_layout_passes=False),
    )
    out_t = fn(tables_tr, xidx3)
    return out_t.T
